# baseline (speedup 1.0000x reference)
import functools

import jax
import jax.numpy as jnp
from jax import lax
from jax.experimental import pallas as pl
from jax.experimental.pallas import tpu as pltpu
from jax.experimental.pallas import tpu_sc as plsc

N_CORES = 2
N_SUBCORES = 16
LANES = 16
CHUNK = 64
HALF = 5120
TRASH = HALF
ACC_R = HALF + 8

_GDN = lax.GatherDimensionNumbers(
    offset_dims=(), collapsed_slice_dims=(0,), start_index_map=(0,))


def _bcast(v16, lane):
    idx = jnp.full((LANES, 1), lane, dtype=jnp.int32)
    return lax.gather(v16, idx, _GDN, (1,),
                      mode=lax.GatherScatterMode.PROMISE_IN_BOUNDS)


def _make_sc_aggregate(n_xpad, d, n_chunks):
    x_rows_per_tile = n_xpad // N_SUBCORES
    out_rows_per_tile = HALF // N_SUBCORES
    mesh = plsc.VectorSubcoreMesh(
        core_axis_name="c", subcore_axis_name="s",
        num_cores=N_CORES, num_subcores=N_SUBCORES)

    @functools.partial(
        pl.kernel,
        out_type=jax.ShapeDtypeStruct((N_CORES, HALF, d), jnp.float32),
        mesh=mesh,
        scratch_types=[
            pltpu.VMEM((CHUNK, d), jnp.float32),
            pltpu.VMEM((2, 2, CHUNK), jnp.int32),
            pltpu.VMEM((2, CHUNK), jnp.int32),
            pltpu.SemaphoreType.DMA,
            pltpu.SemaphoreType.DMA,
            pltpu.SemaphoreType.DMA,
            pltpu.SemaphoreType.DMA,
            pltpu.VMEM_SHARED((n_xpad, d), jnp.float32),
            pltpu.VMEM_SHARED((ACC_R, d), jnp.float32),
        ],
    )
    def sc_aggregate(x_hbm, meta_hbm, dst_hbm, zero_hbm, out_hbm,
                     rows0, meta_v, dst_v, ms0, ms1, ds0, ds1,
                     x_sp, acc):
        msem = [ms0, ms1]
        dsem = [ds0, ds1]

        cid = lax.axis_index("c")
        sid = lax.axis_index("s")

        xr0 = sid * x_rows_per_tile
        pltpu.sync_copy(x_hbm.at[pl.ds(xr0, x_rows_per_tile)],
                        x_sp.at[pl.ds(xr0, x_rows_per_tile)])
        ar0 = sid * out_rows_per_tile
        pltpu.sync_copy(zero_hbm.at[pl.ds(ar0, out_rows_per_tile)],
                        acc.at[pl.ds(ar0, out_rows_per_tile)])
        plsc.subcore_barrier()

        base = sid * n_chunks

        def gc(c):
            return base + lax.rem(c, n_chunks)

        def start_fetch(c, q):
            pltpu.make_async_copy(meta_hbm.at[gc(c)], meta_v.at[q],
                                  msem[q]).start()
            pltpu.make_async_copy(dst_hbm.at[cid, gc(c)], dst_v.at[q],
                                  dsem[q]).start()

        def wait_dst(c, q):
            pltpu.make_async_copy(dst_hbm.at[cid, gc(c)], dst_v.at[q],
                                  dsem[q]).wait()

        def wait_meta(c, q):
            pltpu.make_async_copy(meta_hbm.at[gc(c)], meta_v.at[q],
                                  msem[q]).wait()

        def sync_g(q):
            pltpu.sync_copy(x_sp.at[meta_v.at[q, 0]], rows0)

        def scale(q):
            rb = rows0
            for g in range(CHUNK // LANES):
                v16 = lax.bitcast_convert_type(
                    meta_v[q, 1, pl.ds(g * LANES, LANES)], jnp.float32)
                for l in range(LANES):
                    e = g * LANES + l
                    bc = _bcast(v16, l)
                    for k in range(d // LANES):
                        sl = pl.ds(k * LANES, LANES)
                        rb[e, sl] = rb[e, sl] * bc

        start_fetch(0, 0)
        start_fetch(1, 1)

        def slot(c, q):
            wait_meta(c, q)
            sync_g(q)
            scale(q)
            wait_dst(c, q)
            pltpu.sync_copy(rows0, acc.at[dst_v.at[q]], add=True)
            start_fetch(c + 2, q)

        def pair(p, carry):
            c = 2 * p
            slot(c, 0)
            slot(c + 1, 1)
            return carry

        lax.fori_loop(0, n_chunks // 2, pair, 0)

        wait_meta(n_chunks, 0)
        wait_dst(n_chunks, 0)
        wait_meta(n_chunks + 1, 1)
        wait_dst(n_chunks + 1, 1)

        plsc.subcore_barrier()

        pltpu.sync_copy(acc.at[pl.ds(ar0, out_rows_per_tile)],
                        out_hbm.at[cid, pl.ds(ar0, out_rows_per_tile)])

    return sc_aggregate


def _tc_combine_matmul(partials, w, n_nodes):
    d = partials.shape[2]

    def body(p_ref, w_ref, o_ref):
        agg = jnp.concatenate(
            [p_ref[0], p_ref[1, :n_nodes - HALF]], axis=0)
        o_ref[...] = lax.dot_general(
            agg, w_ref[...], (((1,), (1,)), ((), ())),
            preferred_element_type=jnp.float32)

    return pl.pallas_call(
        body,
        out_shape=jax.ShapeDtypeStruct((n_nodes, d), jnp.float32),
    )(partials, w)


def kernel(_x, adj_indices, adj_values, W):
    n_nodes, d = _x.shape
    e = adj_values.shape[0]

    row = adj_indices[0].astype(jnp.int32)
    col = adj_indices[1].astype(jnp.int32)
    val = adj_values.astype(jnp.float32)

    per_tile = -(-e // (N_SUBCORES * CHUNK * 2)) * CHUNK * 2
    n_chunks = per_tile // CHUNK
    pad = per_tile * N_SUBCORES - e
    if pad:
        row = jnp.concatenate([row, jnp.zeros((pad,), jnp.int32)])
        col = jnp.concatenate([col, jnp.zeros((pad,), jnp.int32)])
        val = jnp.concatenate([val, jnp.zeros((pad,), jnp.float32)])

    vbits = lax.bitcast_convert_type(val, jnp.int32)
    meta = jnp.stack([col.reshape(-1, CHUNK), vbits.reshape(-1, CHUNK)],
                     axis=1)
    dst0 = jnp.where(row < HALF, row, TRASH).reshape(-1, CHUNK)
    dst1 = jnp.where(row >= HALF, row - HALF, TRASH).reshape(-1, CHUNK)
    dsts = jnp.stack([dst0, dst1], axis=0)

    n_xpad = -(-n_nodes // (8 * N_SUBCORES)) * 8 * N_SUBCORES
    x_p = jnp.concatenate(
        [_x, jnp.zeros((n_xpad - n_nodes, d), jnp.float32)])
    zero = jnp.zeros((HALF, d), jnp.float32)

    sc_aggregate = _make_sc_aggregate(n_xpad, d, n_chunks)
    partials = sc_aggregate(x_p, meta, dsts, zero)
    return _tc_combine_matmul(partials, W, n_nodes)


# speedup vs baseline: 1.0197x; 1.0197x over previous
import functools

import jax
import jax.numpy as jnp
from jax import lax
from jax.experimental import pallas as pl
from jax.experimental.pallas import tpu as pltpu
from jax.experimental.pallas import tpu_sc as plsc

N_CORES = 2
N_SUBCORES = 16
LANES = 16
CHUNK = 64
HALF = 5120
TRASH = HALF
ACC_R = HALF + 8

_GDN = lax.GatherDimensionNumbers(
    offset_dims=(), collapsed_slice_dims=(0,), start_index_map=(0,))


def _bcast(v16, lane):
    idx = jnp.full((LANES, 1), lane, dtype=jnp.int32)
    return lax.gather(v16, idx, _GDN, (1,),
                      mode=lax.GatherScatterMode.PROMISE_IN_BOUNDS)


def _make_sc_aggregate(n_xpad, d, n_chunks):
    x_rows_per_tile = n_xpad // N_SUBCORES
    out_rows_per_tile = HALF // N_SUBCORES
    mesh = plsc.VectorSubcoreMesh(
        core_axis_name="c", subcore_axis_name="s",
        num_cores=N_CORES, num_subcores=N_SUBCORES)

    @functools.partial(
        pl.kernel,
        out_type=jax.ShapeDtypeStruct((N_CORES, HALF, d), jnp.float32),
        mesh=mesh,
        scratch_types=[
            pltpu.VMEM((CHUNK, d), jnp.float32),
            pltpu.VMEM((2, CHUNK), jnp.int32),
            pltpu.VMEM((2, CHUNK), jnp.float32),
            pltpu.VMEM((2, CHUNK), jnp.int32),
            pltpu.SemaphoreType.DMA,
            pltpu.SemaphoreType.DMA,
            pltpu.SemaphoreType.DMA,
            pltpu.SemaphoreType.DMA,
            pltpu.SemaphoreType.DMA,
            pltpu.SemaphoreType.DMA,
            pltpu.VMEM_SHARED((n_xpad, d), jnp.float32),
            pltpu.VMEM_SHARED((ACC_R, d), jnp.float32),
        ],
    )
    def sc_aggregate(x_hbm, col_hbm, val_hbm, dst_hbm, zero_hbm, out_hbm,
                     rows0, col_v, val_v, dst_v,
                     cs0, cs1, vs0, vs1, ds0, ds1,
                     x_sp, acc):
        csem = [cs0, cs1]
        vsem = [vs0, vs1]
        dsem = [ds0, ds1]

        cid = lax.axis_index("c")
        sid = lax.axis_index("s")

        xr0 = sid * x_rows_per_tile
        pltpu.sync_copy(x_hbm.at[pl.ds(xr0, x_rows_per_tile)],
                        x_sp.at[pl.ds(xr0, x_rows_per_tile)])
        ar0 = sid * out_rows_per_tile
        pltpu.sync_copy(zero_hbm.at[pl.ds(ar0, out_rows_per_tile)],
                        acc.at[pl.ds(ar0, out_rows_per_tile)])
        plsc.subcore_barrier()

        base = sid * n_chunks

        def gc(c):
            return base + lax.rem(c, n_chunks)

        def start_fetch(c, q):
            pltpu.make_async_copy(col_hbm.at[gc(c)], col_v.at[q],
                                  csem[q]).start()
            pltpu.make_async_copy(val_hbm.at[gc(c)], val_v.at[q],
                                  vsem[q]).start()
            pltpu.make_async_copy(dst_hbm.at[cid, gc(c)], dst_v.at[q],
                                  dsem[q]).start()

        def wait_col(c, q):
            pltpu.make_async_copy(col_hbm.at[gc(c)], col_v.at[q],
                                  csem[q]).wait()

        def wait_valdst(c, q):
            pltpu.make_async_copy(val_hbm.at[gc(c)], val_v.at[q],
                                  vsem[q]).wait()
            pltpu.make_async_copy(dst_hbm.at[cid, gc(c)], dst_v.at[q],
                                  dsem[q]).wait()

        def sync_g(q):
            pltpu.sync_copy(x_sp.at[col_v.at[q]], rows0)

        def scale(q):
            rb = rows0
            for g in range(CHUNK // LANES):
                v16 = val_v[q, pl.ds(g * LANES, LANES)]
                for l in range(LANES):
                    e = g * LANES + l
                    bc = _bcast(v16, l)
                    for k in range(d // LANES):
                        sl = pl.ds(k * LANES, LANES)
                        rb[e, sl] = rb[e, sl] * bc

        start_fetch(0, 0)
        start_fetch(1, 1)

        def slot(c, q):
            wait_col(c, q)
            sync_g(q)
            wait_valdst(c, q)
            scale(q)
            pltpu.sync_copy(rows0, acc.at[dst_v.at[q]], add=True)
            start_fetch(c + 2, q)

        def pair(p, carry):
            c = 2 * p
            slot(c, 0)
            slot(c + 1, 1)
            return carry

        lax.fori_loop(0, n_chunks // 2, pair, 0)

        wait_col(n_chunks, 0)
        wait_valdst(n_chunks, 0)
        wait_col(n_chunks + 1, 1)
        wait_valdst(n_chunks + 1, 1)

        plsc.subcore_barrier()

        pltpu.sync_copy(acc.at[pl.ds(ar0, out_rows_per_tile)],
                        out_hbm.at[cid, pl.ds(ar0, out_rows_per_tile)])

    return sc_aggregate


def _tc_combine_matmul(partials, w, n_nodes):
    d = partials.shape[2]

    def body(p_ref, w_ref, o_ref):
        agg = jnp.concatenate(
            [p_ref[0], p_ref[1, :n_nodes - HALF]], axis=0)
        o_ref[...] = lax.dot_general(
            agg, w_ref[...], (((1,), (1,)), ((), ())),
            preferred_element_type=jnp.float32)

    return pl.pallas_call(
        body,
        out_shape=jax.ShapeDtypeStruct((n_nodes, d), jnp.float32),
    )(partials, w)


def kernel(_x, adj_indices, adj_values, W):
    n_nodes, d = _x.shape
    e = adj_values.shape[0]

    row = adj_indices[0].astype(jnp.int32)
    col = adj_indices[1].astype(jnp.int32)
    val = adj_values.astype(jnp.float32)

    per_tile = -(-e // (N_SUBCORES * CHUNK * 2)) * CHUNK * 2
    n_chunks = per_tile // CHUNK
    pad = per_tile * N_SUBCORES - e
    if pad:
        row = jnp.concatenate([row, jnp.zeros((pad,), jnp.int32)])
        col = jnp.concatenate([col, jnp.zeros((pad,), jnp.int32)])
        val = jnp.concatenate([val, jnp.zeros((pad,), jnp.float32)])

    col2 = col.reshape(-1, CHUNK)
    val2 = val.reshape(-1, CHUNK)
    dst0 = jnp.where(row < HALF, row, TRASH).reshape(-1, CHUNK)
    dst1 = jnp.where(row >= HALF, row - HALF, TRASH).reshape(-1, CHUNK)
    dsts = jnp.stack([dst0, dst1], axis=0)

    n_xpad = -(-n_nodes // (8 * N_SUBCORES)) * 8 * N_SUBCORES
    x_p = jnp.concatenate(
        [_x, jnp.zeros((n_xpad - n_nodes, d), jnp.float32)])
    zero = jnp.zeros((HALF, d), jnp.float32)

    sc_aggregate = _make_sc_aggregate(n_xpad, d, n_chunks)
    partials = sc_aggregate(x_p, col2, val2, dsts, zero)
    return _tc_combine_matmul(partials, W, n_nodes)
